# baseline (speedup 1.0000x reference)
import functools

import jax
import jax.numpy as jnp
from jax import lax
from jax.experimental import pallas as pl
from jax.experimental.pallas import tpu as pltpu
from jax.experimental.pallas import tpu_sc as plsc

N_NODES = 10000
E_EDGES = 160000
N_LBL = 20000

NUM_TILES = 16
EPAD = 163840
EPT = EPAD // NUM_TILES
NB = EPT // 128
RPAD = 10240
NPAD = RPAD
RPT = RPAD // NUM_TILES
LPAD = 20480
LPT = LPAD // NUM_TILES
LNB = LPT // 128

_MESH = dict(core_axis_name="c", subcore_axis_name="s")


NHB = NB // 2


def _agg_side(sidx2d, didx2d, tables, outs, zeros_hbm, acc, sidx_v, didx_v,
              rows_a, rows_b, g_a, g_b, s_a, s_b, s):
    for tab, out in zip(tables, outs):
        for k in range(RPT // 128):
            pltpu.sync_copy(zeros_hbm, acc.at[pl.ds(s * RPT + k * 128, 128)])
        plsc.subcore_barrier()
        for h in range(2):
            pltpu.sync_copy(sidx2d.at[pl.ds(s * NB + h * NHB, NHB)], sidx_v)
            pltpu.sync_copy(didx2d.at[pl.ds(s * NB + h * NHB, NHB)], didx_v)
            pltpu.async_copy(tab.at[sidx_v.at[0]], rows_a, g_a)

            def grp(g, carry):
                for b in range(2):
                    if b == 0:
                        rows_cur, g_cur, s_cur = rows_a, g_a, s_a
                        rows_nxt, g_nxt, s_nxt = rows_b, g_b, s_b
                    else:
                        rows_cur, g_cur, s_cur = rows_b, g_b, s_b
                        rows_nxt, g_nxt, s_nxt = rows_a, g_a, s_a
                    i = g * 2 + b
                    inext = i + 1
                    pltpu.make_async_copy(tab.at[sidx_v.at[0]], rows_cur, g_cur).wait()
                    pltpu.async_copy(rows_cur, acc.at[didx_v.at[i]], s_cur, add=True)

                    if b == 0:
                        @pl.when((inext < NHB) & (g >= 1))
                        def _(rows_nxt=rows_nxt, s_nxt=s_nxt):
                            pltpu.make_async_copy(
                                rows_nxt, acc.at[didx_v.at[0]], s_nxt).wait()
                    else:
                        @pl.when(inext < NHB)
                        def _(rows_nxt=rows_nxt, s_nxt=s_nxt):
                            pltpu.make_async_copy(
                                rows_nxt, acc.at[didx_v.at[0]], s_nxt).wait()

                    @pl.when(inext < NHB)
                    def _(inext=inext, rows_nxt=rows_nxt, g_nxt=g_nxt):
                        pltpu.async_copy(tab.at[sidx_v.at[inext]], rows_nxt, g_nxt)
                return carry

            lax.fori_loop(0, NHB // 2, grp, 0)
            pltpu.make_async_copy(rows_a, acc.at[didx_v.at[0]], s_a).wait()
            pltpu.make_async_copy(rows_b, acc.at[didx_v.at[0]], s_b).wait()
        plsc.subcore_barrier()
        for k in range(5):
            r = s * RPT + k * 128
            pltpu.sync_copy(acc.at[pl.ds(r, 128)], rows_a)
            pltpu.sync_copy(rows_a, out.at[pl.ds(r, 128)])
        plsc.subcore_barrier()


def _make_layer_agg(nch):
    mesh = plsc.VectorSubcoreMesh(**_MESH)
    out_type = tuple(jax.ShapeDtypeStruct((NPAD, 128), jnp.float32)
                     for _ in range(2 * nch))
    scratch = [
        pltpu.VMEM_SHARED((RPAD, 128), jnp.float32),
        pltpu.VMEM((NHB, 128), jnp.int32),
        pltpu.VMEM((NHB, 128), jnp.int32),
        pltpu.VMEM((128, 128), jnp.float32),
        pltpu.VMEM((128, 128), jnp.float32),
        pltpu.SemaphoreType.DMA,
        pltpu.SemaphoreType.DMA,
        pltpu.SemaphoreType.DMA,
        pltpu.SemaphoreType.DMA,
    ]

    @functools.partial(pl.kernel, out_type=out_type, mesh=mesh,
                       scratch_types=scratch)
    def k(s_tw, d_tw, s_wt, d_wt, *rest):
        tabs_t = rest[0:nch]
        tabs_w = rest[nch:2 * nch]
        zeros_hbm = rest[2 * nch]
        outs_w = rest[2 * nch + 1: 2 * nch + 1 + nch]
        outs_t = rest[2 * nch + 1 + nch: 2 * nch + 1 + 2 * nch]
        (acc, sidx_v, didx_v, rows_a, rows_b, g_a, g_b, s_a, s_b) = \
            rest[2 * nch + 1 + 2 * nch:]
        c = lax.axis_index("c")
        s = lax.axis_index("s")

        @pl.when(c == 0)
        def _():
            _agg_side(s_tw, d_tw, tabs_t, outs_w, zeros_hbm, acc, sidx_v,
                      didx_v, rows_a, rows_b, g_a, g_b, s_a, s_b, s)

        @pl.when(c == 1)
        def _():
            _agg_side(s_wt, d_wt, tabs_w, outs_t, zeros_hbm, acc, sidx_v,
                      didx_v, rows_a, rows_b, g_a, g_b, s_a, s_b, s)

    return k


def _make_count_kernel():
    mesh = plsc.VectorSubcoreMesh(**_MESH)
    out_type = (jax.ShapeDtypeStruct((RPAD,), jnp.float32),
                jax.ShapeDtypeStruct((RPAD,), jnp.float32))
    scratch = [
        pltpu.VMEM_SHARED((RPAD,), jnp.float32),
        pltpu.VMEM((NB, 128), jnp.int32),
        pltpu.VMEM((128,), jnp.float32),
        pltpu.VMEM((RPT,), jnp.float32),
    ]

    @functools.partial(pl.kernel, out_type=out_type, mesh=mesh,
                       scratch_types=scratch)
    def k(d_tw, d_wt, ones_hbm, zeros_hbm, out_tw, out_wt, acc1, didx_v,
          ones_v, stage_v):
        c = lax.axis_index("c")
        s = lax.axis_index("s")

        def side(didx2d, out):
            pltpu.sync_copy(zeros_hbm, acc1.at[pl.ds(s * RPT, RPT)])
            plsc.subcore_barrier()
            pltpu.sync_copy(didx2d.at[pl.ds(s * NB, NB)], didx_v)
            pltpu.sync_copy(ones_hbm, ones_v)

            def it(i, carry):
                pltpu.sync_copy(ones_v, acc1.at[didx_v.at[i]], add=True)
                return carry

            lax.fori_loop(0, NB, it, 0)
            plsc.subcore_barrier()
            pltpu.sync_copy(acc1.at[pl.ds(s * RPT, RPT)], stage_v)
            pltpu.sync_copy(stage_v, out.at[pl.ds(s * RPT, RPT)])

        @pl.when(c == 0)
        def _():
            side(d_tw, out_tw)

        @pl.when(c == 1)
        def _():
            side(d_wt, out_wt)

    return k


def _make_lp_gather():
    nch = 4
    mesh = plsc.VectorSubcoreMesh(**_MESH)
    out_type = tuple(jax.ShapeDtypeStruct((LPAD, 128), jnp.float32)
                     for _ in range(2 * nch))
    scratch = [
        pltpu.VMEM((LNB, 128), jnp.int32),
        pltpu.VMEM((128, 128), jnp.float32),
        pltpu.VMEM((128, 128), jnp.float32),
        pltpu.SemaphoreType.DMA,
        pltpu.SemaphoreType.DMA,
        pltpu.SemaphoreType.DMA,
        pltpu.SemaphoreType.DMA,
    ]

    @functools.partial(pl.kernel, out_type=out_type, mesh=mesh,
                       scratch_types=scratch)
    def k(eli0, eli1, *rest):
        tabs_t = rest[0:nch]
        tabs_w = rest[nch:2 * nch]
        outs_s = rest[2 * nch: 3 * nch]
        outs_d = rest[3 * nch: 4 * nch]
        idx_v, rows_a, rows_b, g_a, g_b, w_a, w_b = rest[4 * nch:]
        c = lax.axis_index("c")
        s = lax.axis_index("s")

        def side(eli, tabs, outs):
            pltpu.sync_copy(eli.at[s], idx_v)
            for tab, out in zip(tabs, outs):
                pltpu.async_copy(tab.at[idx_v.at[0]], rows_a, g_a)

                def grp(g, carry, tab=tab, out=out):
                    for b in range(2):
                        if b == 0:
                            rows_cur, g_cur, w_cur = rows_a, g_a, w_a
                            rows_nxt, g_nxt, w_nxt = rows_b, g_b, w_b
                        else:
                            rows_cur, g_cur, w_cur = rows_b, g_b, w_b
                            rows_nxt, g_nxt, w_nxt = rows_a, g_a, w_a
                        i = g * 2 + b
                        inext = i + 1
                        pltpu.make_async_copy(tab.at[idx_v.at[0]], rows_cur, g_cur).wait()
                        pltpu.async_copy(
                            rows_cur, out.at[pl.ds(s * LPT + i * 128, 128)], w_cur)

                        if b == 0:
                            @pl.when((inext < LNB) & (g >= 1))
                            def _(rows_nxt=rows_nxt, w_nxt=w_nxt):
                                pltpu.make_async_copy(
                                    rows_nxt, out.at[pl.ds(s * LPT, 128)], w_nxt).wait()
                        else:
                            @pl.when(inext < LNB)
                            def _(rows_nxt=rows_nxt, w_nxt=w_nxt):
                                pltpu.make_async_copy(
                                    rows_nxt, out.at[pl.ds(s * LPT, 128)], w_nxt).wait()

                        @pl.when(inext < LNB)
                        def _(inext=inext, rows_nxt=rows_nxt, g_nxt=g_nxt):
                            pltpu.async_copy(tab.at[idx_v.at[inext]], rows_nxt, g_nxt)
                    return carry

                lax.fori_loop(0, LNB // 2, grp, 0)
                pltpu.make_async_copy(rows_a, out.at[pl.ds(s * LPT, 128)], w_a).wait()
                pltpu.make_async_copy(rows_b, out.at[pl.ds(s * LPT, 128)], w_b).wait()

        @pl.when(c == 0)
        def _():
            side(eli0, tabs_t, outs_s)

        @pl.when(c == 1)
        def _():
            side(eli1, tabs_w, outs_d)

    return k


def _conv_body(nk, nj, nm, *refs):
    aggs = refs[0:nk]
    cnt_ref = refs[nk]
    hs = refs[nk + 1: nk + 1 + nj]
    wa_ref = refs[nk + 1 + nj]
    wr_ref = refs[nk + 2 + nj]
    b_ref = refs[nk + 3 + nj]
    outs = refs[nk + 4 + nj:]
    inv = 1.0 / jnp.maximum(cnt_ref[...], 1.0)
    acc = jnp.broadcast_to(b_ref[...], (aggs[0].shape[0], b_ref.shape[1]))
    for j, a in enumerate(aggs):
        acc = acc + jnp.dot(a[...] * inv, wa_ref[pl.ds(j * 128, 128), :],
                            preferred_element_type=jnp.float32)
    for j, h in enumerate(hs):
        acc = acc + jnp.dot(h[...], wr_ref[pl.ds(j * 128, 128), :],
                            preferred_element_type=jnp.float32)
    for m, o in enumerate(outs):
        o[...] = acc[:, m * 128:(m + 1) * 128]


def _conv_tc(agg_chunks, cnt, h_chunks, Wa, Wr, b):
    nk, nj = len(agg_chunks), len(h_chunks)
    dout = Wa.shape[1]
    nm = dout // 128
    bn = 1024
    grid = (NPAD // bn,)
    in_specs = (
        [pl.BlockSpec((bn, 128), lambda i: (i, 0)) for _ in range(nk)]
        + [pl.BlockSpec((bn, 1), lambda i: (i, 0))]
        + [pl.BlockSpec((bn, 128), lambda i: (i, 0)) for _ in range(nj)]
        + [pl.BlockSpec((Wa.shape[0], dout), lambda i: (0, 0)),
           pl.BlockSpec((Wr.shape[0], dout), lambda i: (0, 0)),
           pl.BlockSpec((1, dout), lambda i: (0, 0))]
    )
    out_specs = [pl.BlockSpec((bn, 128), lambda i: (i, 0)) for _ in range(nm)]
    out_shape = [jax.ShapeDtypeStruct((NPAD, 128), jnp.float32)
                 for _ in range(nm)]
    return pl.pallas_call(
        functools.partial(_conv_body, nk, nj, nm),
        grid=grid,
        in_specs=in_specs,
        out_specs=out_specs,
        out_shape=out_shape,
    )(*agg_chunks, cnt, *h_chunks, Wa, Wr, b[None, :])


def _proj_body(nj, nm, *refs):
    hs = refs[0:nj]
    w_ref = refs[nj]
    outs = refs[nj + 1:]
    acc = jnp.dot(hs[0][...], w_ref[pl.ds(0, 128), :],
                  preferred_element_type=jnp.float32)
    for j in range(1, nj):
        acc = acc + jnp.dot(hs[j][...], w_ref[pl.ds(j * 128, 128), :],
                            preferred_element_type=jnp.float32)
    for m, o in enumerate(outs):
        o[...] = acc[:, m * 128:(m + 1) * 128]


def _proj_tc(h_chunks, W):
    nj = len(h_chunks)
    dout = W.shape[1]
    nm = dout // 128
    bn = 1024
    in_specs = ([pl.BlockSpec((bn, 128), lambda i: (i, 0)) for _ in range(nj)]
                + [pl.BlockSpec((W.shape[0], dout), lambda i: (0, 0))])
    return pl.pallas_call(
        functools.partial(_proj_body, nj, nm),
        grid=(NPAD // bn,),
        in_specs=in_specs,
        out_specs=[pl.BlockSpec((bn, 128), lambda i: (i, 0)) for _ in range(nm)],
        out_shape=[jax.ShapeDtypeStruct((NPAD, 128), jnp.float32)
                   for _ in range(nm)],
    )(*h_chunks, W)


def _conv_pre_body(nk, nj, *refs):
    aggs = refs[0:nk]
    cnt_ref = refs[nk]
    hs = refs[nk + 1: nk + 1 + nj]
    wr_ref = refs[nk + 1 + nj]
    b_ref = refs[nk + 2 + nj]
    outs = refs[nk + 3 + nj:]
    inv = 1.0 / jnp.maximum(cnt_ref[...], 1.0)
    acc = jnp.broadcast_to(b_ref[...], (hs[0].shape[0], b_ref.shape[1]))
    for j, h in enumerate(hs):
        acc = acc + jnp.dot(h[...], wr_ref[pl.ds(j * 128, 128), :],
                            preferred_element_type=jnp.float32)
    for m, o in enumerate(outs):
        o[...] = aggs[m][...] * inv + acc[:, m * 128:(m + 1) * 128]


def _conv_pre_tc(agg_chunks, cnt, h_chunks, Wr, b):
    nk, nj = len(agg_chunks), len(h_chunks)
    dout = Wr.shape[1]
    nm = dout // 128
    assert nk == nm
    bn = 1024
    in_specs = (
        [pl.BlockSpec((bn, 128), lambda i: (i, 0)) for _ in range(nk)]
        + [pl.BlockSpec((bn, 1), lambda i: (i, 0))]
        + [pl.BlockSpec((bn, 128), lambda i: (i, 0)) for _ in range(nj)]
        + [pl.BlockSpec((Wr.shape[0], dout), lambda i: (0, 0)),
           pl.BlockSpec((1, dout), lambda i: (0, 0))]
    )
    return pl.pallas_call(
        functools.partial(_conv_pre_body, nk, nj),
        grid=(NPAD // bn,),
        in_specs=in_specs,
        out_specs=[pl.BlockSpec((bn, 128), lambda i: (i, 0)) for _ in range(nm)],
        out_shape=[jax.ShapeDtypeStruct((NPAD, 128), jnp.float32)
                   for _ in range(nm)],
    )(*agg_chunks, cnt, *h_chunks, Wr, b[None, :])


def _linkpred_body(s0, s1, s2, s3, d0, d1, d2, d3, o_ref):
    acc = jnp.sum(s0[...] * d0[...], axis=-1)
    acc += jnp.sum(s1[...] * d1[...], axis=-1)
    acc += jnp.sum(s2[...] * d2[...], axis=-1)
    acc += jnp.sum(s3[...] * d3[...], axis=-1)
    o_ref[...] = jax.nn.sigmoid(acc)


def _linkpred_tc(s_chunks, d_chunks):
    bl = 2048
    spec = pl.BlockSpec((bl, 128), lambda i: (i, 0))
    out = pl.pallas_call(
        _linkpred_body,
        grid=(LPAD // bl,),
        in_specs=[spec] * 8,
        out_specs=pl.BlockSpec((bl,), lambda i: (i,)),
        out_shape=jax.ShapeDtypeStruct((LPAD,), jnp.float32),
    )(*s_chunks, *d_chunks)
    return out[:N_LBL]


def _pad_edges(ei):
    pad = EPAD - E_EDGES
    sidx = jnp.concatenate([ei[0], jnp.arange(pad, dtype=jnp.int32) % N_NODES])
    didx = jnp.concatenate(
        [ei[1], N_NODES + (jnp.arange(pad, dtype=jnp.int32) % 128)])
    return sidx.reshape(EPAD // 128, 128), didx.reshape(EPAD // 128, 128)


def kernel(x_transactions, x_wallets, enc0_Wa, enc0_Wr, enc0_b, enc1_Wa, enc1_Wr, enc1_b,
           dec0_Wa, dec0_Wr, dec0_b, last_Wa, last_Wr, last_b,
           edge_index_tw, edge_index_wt, edge_label_index):
    s_tw, d_tw = _pad_edges(edge_index_tw)
    s_wt, d_wt = _pad_edges(edge_index_wt)
    zeros128 = jnp.zeros((128, 128), jnp.float32)
    ones128 = jnp.ones((128,), jnp.float32)
    zeros640 = jnp.zeros((RPT,), jnp.float32)

    cnt_tw_p, cnt_wt_p = _make_count_kernel()(d_tw, d_wt, ones128, zeros640)
    cnt_tw = cnt_tw_p[:, None]
    cnt_wt = cnt_wt_p[:, None]

    xt_pad = jnp.pad(x_transactions, ((0, NPAD - N_NODES), (0, 0)))
    xw_pad = jnp.pad(x_wallets, ((0, NPAD - N_NODES), (0, 0)))
    ht = [xt_pad[:, i * 128:(i + 1) * 128] for i in range(2)]
    hw = [xw_pad[:, i * 128:(i + 1) * 128] for i in range(2)]

    agg2 = _make_layer_agg(2)
    agg4 = _make_layer_agg(4)

    def layer(ht, hw, Wa, Wr, b):
        aggk = agg2 if len(ht) == 2 else agg4
        res = aggk(s_tw, d_tw, s_wt, d_wt, *ht, *hw, zeros128)
        nch = len(ht)
        aw, at = list(res[:nch]), list(res[nch:])
        new_w = _conv_tc(aw, cnt_tw, hw, Wa[0], Wr[0], b[0])
        new_t = _conv_tc(at, cnt_wt, ht, Wa[1], Wr[1], b[1])
        return list(new_t), list(new_w)

    ht, hw = layer(ht, hw, enc0_Wa, enc0_Wr, enc0_b)
    ht, hw = layer(ht, hw, enc1_Wa, enc1_Wr, enc1_b)
    hid_t_c, hid_w_c = ht, hw
    ft, fw = layer(ht, hw, dec0_Wa, dec0_Wr, dec0_b)
    pt = _proj_tc(ft, last_Wa[0])
    pw = _proj_tc(fw, last_Wa[1])
    res = agg2(s_tw, d_tw, s_wt, d_wt, *pt, *pw, zeros128)
    aw, at = list(res[:2]), list(res[2:])
    new_w = _conv_pre_tc(aw, cnt_tw, fw, last_Wr[0], last_b[0])
    new_t = _conv_pre_tc(at, cnt_wt, ft, last_Wr[1], last_b[1])
    ft, fw = list(new_t), list(new_w)

    eli0 = jnp.concatenate(
        [edge_label_index[0], jnp.zeros((LPAD - N_LBL,), jnp.int32)]
    ).reshape(NUM_TILES, LNB, 128)
    eli1 = jnp.concatenate(
        [edge_label_index[1], jnp.zeros((LPAD - N_LBL,), jnp.int32)]
    ).reshape(NUM_TILES, LNB, 128)
    lp = _make_lp_gather()(eli0, eli1, *hid_t_c, *hid_w_c)
    s_chunks, d_chunks = lp[:4], lp[4:]
    edge_pred = _linkpred_tc(s_chunks, d_chunks)

    hid_t = jnp.concatenate(hid_t_c, axis=1)[:N_NODES]
    hid_w = jnp.concatenate(hid_w_c, axis=1)[:N_NODES]
    f_t = jnp.concatenate(ft, axis=1)[:N_NODES]
    f_w = jnp.concatenate(fw, axis=1)[:N_NODES]
    return (hid_t, hid_w, f_t, f_w, edge_pred)


# speedup vs baseline: 1.1448x; 1.1448x over previous
import functools

import jax
import jax.numpy as jnp
from jax import lax
from jax.experimental import pallas as pl
from jax.experimental.pallas import tpu as pltpu
from jax.experimental.pallas import tpu_sc as plsc

N_NODES = 10000
E_EDGES = 160000
N_LBL = 20000

NUM_TILES = 16
EPAD = 163840
EPT = EPAD // NUM_TILES
NB = EPT // 128
RPAD = 10240
NPAD = RPAD
RPT = RPAD // NUM_TILES
LPAD = 20480
LPT = LPAD // NUM_TILES
LNB = LPT // 128

_MESH = dict(core_axis_name="c", subcore_axis_name="s")


NHB = NB // 2


def _agg_side(sidx2d, didx2d, tables, outs, zeros_hbm, acc, sidx_v, didx_v,
              rows_a, rows_b, g_a, g_b, s_a, s_b, s):
    for tab, out in zip(tables, outs):
        for k in range(RPT // 128):
            pltpu.sync_copy(zeros_hbm, acc.at[pl.ds(s * RPT + k * 128, 128)])
        plsc.subcore_barrier()
        for h in range(2):
            pltpu.sync_copy(sidx2d.at[pl.ds(s * NB + h * NHB, NHB)], sidx_v)
            pltpu.sync_copy(didx2d.at[pl.ds(s * NB + h * NHB, NHB)], didx_v)
            pltpu.async_copy(tab.at[sidx_v.at[0]], rows_a, g_a)

            def grp(g, carry):
                for b in range(2):
                    rows_cur, sem_cur = (rows_a, g_a) if b == 0 else (rows_b, g_b)
                    rows_nxt, sem_nxt = (rows_b, g_b) if b == 0 else (rows_a, g_a)
                    i = g * 2 + b
                    inext = i + 1

                    @pl.when(inext < NHB)
                    def _(inext=inext, rows_nxt=rows_nxt, sem_nxt=sem_nxt):
                        pltpu.async_copy(tab.at[sidx_v.at[inext]], rows_nxt, sem_nxt)

                    pltpu.make_async_copy(tab.at[sidx_v.at[0]], rows_cur, sem_cur).wait()
                    pltpu.sync_copy(rows_cur, acc.at[didx_v.at[i]], add=True)
                return carry

            lax.fori_loop(0, NHB // 2, grp, 0)
        plsc.subcore_barrier()
        for k in range(5):
            r = s * RPT + k * 128
            pltpu.sync_copy(acc.at[pl.ds(r, 128)], rows_a)
            pltpu.sync_copy(rows_a, out.at[pl.ds(r, 128)])
        plsc.subcore_barrier()


def _make_layer_agg(nch):
    mesh = plsc.VectorSubcoreMesh(**_MESH)
    out_type = tuple(jax.ShapeDtypeStruct((NPAD, 128), jnp.float32)
                     for _ in range(2 * nch))
    scratch = [
        pltpu.VMEM_SHARED((RPAD, 128), jnp.float32),
        pltpu.VMEM((NHB, 128), jnp.int32),
        pltpu.VMEM((NHB, 128), jnp.int32),
        pltpu.VMEM((128, 128), jnp.float32),
        pltpu.VMEM((128, 128), jnp.float32),
        pltpu.SemaphoreType.DMA,
        pltpu.SemaphoreType.DMA,
        pltpu.SemaphoreType.DMA,
        pltpu.SemaphoreType.DMA,
    ]

    @functools.partial(pl.kernel, out_type=out_type, mesh=mesh,
                       scratch_types=scratch)
    def k(s_tw, d_tw, s_wt, d_wt, *rest):
        tabs_t = rest[0:nch]
        tabs_w = rest[nch:2 * nch]
        zeros_hbm = rest[2 * nch]
        outs_w = rest[2 * nch + 1: 2 * nch + 1 + nch]
        outs_t = rest[2 * nch + 1 + nch: 2 * nch + 1 + 2 * nch]
        (acc, sidx_v, didx_v, rows_a, rows_b, g_a, g_b, s_a, s_b) = \
            rest[2 * nch + 1 + 2 * nch:]
        c = lax.axis_index("c")
        s = lax.axis_index("s")

        @pl.when(c == 0)
        def _():
            _agg_side(s_tw, d_tw, tabs_t, outs_w, zeros_hbm, acc, sidx_v,
                      didx_v, rows_a, rows_b, g_a, g_b, s_a, s_b, s)

        @pl.when(c == 1)
        def _():
            _agg_side(s_wt, d_wt, tabs_w, outs_t, zeros_hbm, acc, sidx_v,
                      didx_v, rows_a, rows_b, g_a, g_b, s_a, s_b, s)

    return k


def _make_count_kernel():
    mesh = plsc.VectorSubcoreMesh(**_MESH)
    out_type = (jax.ShapeDtypeStruct((RPAD,), jnp.float32),
                jax.ShapeDtypeStruct((RPAD,), jnp.float32))
    scratch = [
        pltpu.VMEM_SHARED((RPAD,), jnp.float32),
        pltpu.VMEM((NB, 128), jnp.int32),
        pltpu.VMEM((128,), jnp.float32),
        pltpu.VMEM((RPT,), jnp.float32),
    ]

    @functools.partial(pl.kernel, out_type=out_type, mesh=mesh,
                       scratch_types=scratch)
    def k(d_tw, d_wt, ones_hbm, zeros_hbm, out_tw, out_wt, acc1, didx_v,
          ones_v, stage_v):
        c = lax.axis_index("c")
        s = lax.axis_index("s")

        def side(didx2d, out):
            pltpu.sync_copy(zeros_hbm, acc1.at[pl.ds(s * RPT, RPT)])
            plsc.subcore_barrier()
            pltpu.sync_copy(didx2d.at[pl.ds(s * NB, NB)], didx_v)
            pltpu.sync_copy(ones_hbm, ones_v)

            def it(i, carry):
                pltpu.sync_copy(ones_v, acc1.at[didx_v.at[i]], add=True)
                return carry

            lax.fori_loop(0, NB, it, 0)
            plsc.subcore_barrier()
            pltpu.sync_copy(acc1.at[pl.ds(s * RPT, RPT)], stage_v)
            pltpu.sync_copy(stage_v, out.at[pl.ds(s * RPT, RPT)])

        @pl.when(c == 0)
        def _():
            side(d_tw, out_tw)

        @pl.when(c == 1)
        def _():
            side(d_wt, out_wt)

    return k


def _make_lp_gather():
    nch = 4
    mesh = plsc.VectorSubcoreMesh(**_MESH)
    out_type = tuple(jax.ShapeDtypeStruct((LPAD, 128), jnp.float32)
                     for _ in range(2 * nch))
    scratch = [
        pltpu.VMEM((LNB, 128), jnp.int32),
        pltpu.VMEM((128, 128), jnp.float32),
        pltpu.VMEM((128, 128), jnp.float32),
        pltpu.SemaphoreType.DMA,
        pltpu.SemaphoreType.DMA,
        pltpu.SemaphoreType.DMA,
        pltpu.SemaphoreType.DMA,
    ]

    @functools.partial(pl.kernel, out_type=out_type, mesh=mesh,
                       scratch_types=scratch)
    def k(eli0, eli1, *rest):
        tabs_t = rest[0:nch]
        tabs_w = rest[nch:2 * nch]
        outs_s = rest[2 * nch: 3 * nch]
        outs_d = rest[3 * nch: 4 * nch]
        idx_v, rows_a, rows_b, g_a, g_b, w_a, w_b = rest[4 * nch:]
        c = lax.axis_index("c")
        s = lax.axis_index("s")

        def side(eli, tabs, outs):
            pltpu.sync_copy(eli.at[s], idx_v)
            for tab, out in zip(tabs, outs):
                pltpu.async_copy(tab.at[idx_v.at[0]], rows_a, g_a)

                def grp(g, carry, tab=tab, out=out):
                    for b in range(2):
                        rows_cur, sem_cur = (rows_a, g_a) if b == 0 else (rows_b, g_b)
                        rows_nxt, sem_nxt = (rows_b, g_b) if b == 0 else (rows_a, g_a)
                        i = g * 2 + b
                        inext = i + 1

                        @pl.when(inext < LNB)
                        def _(inext=inext, rows_nxt=rows_nxt, sem_nxt=sem_nxt):
                            pltpu.async_copy(tab.at[idx_v.at[inext]], rows_nxt, sem_nxt)

                        pltpu.make_async_copy(tab.at[idx_v.at[0]], rows_cur, sem_cur).wait()
                        pltpu.sync_copy(rows_cur, out.at[pl.ds(s * LPT + i * 128, 128)])
                    return carry

                lax.fori_loop(0, LNB // 2, grp, 0)

        @pl.when(c == 0)
        def _():
            side(eli0, tabs_t, outs_s)

        @pl.when(c == 1)
        def _():
            side(eli1, tabs_w, outs_d)

    return k


def _conv_body(nk, nj, nm, *refs):
    aggs = refs[0:nk]
    cnt_ref = refs[nk]
    hs = refs[nk + 1: nk + 1 + nj]
    wa_ref = refs[nk + 1 + nj]
    wr_ref = refs[nk + 2 + nj]
    b_ref = refs[nk + 3 + nj]
    outs = refs[nk + 4 + nj:]
    inv = 1.0 / jnp.maximum(cnt_ref[...], 1.0)
    acc = jnp.broadcast_to(b_ref[...], (aggs[0].shape[0], b_ref.shape[1]))
    for j, a in enumerate(aggs):
        acc = acc + jnp.dot(a[...] * inv, wa_ref[pl.ds(j * 128, 128), :],
                            preferred_element_type=jnp.float32)
    for j, h in enumerate(hs):
        acc = acc + jnp.dot(h[...], wr_ref[pl.ds(j * 128, 128), :],
                            preferred_element_type=jnp.float32)
    for m, o in enumerate(outs):
        o[...] = acc[:, m * 128:(m + 1) * 128]


def _conv_tc(agg_chunks, cnt, h_chunks, Wa, Wr, b):
    nk, nj = len(agg_chunks), len(h_chunks)
    dout = Wa.shape[1]
    nm = dout // 128
    bn = 1024
    grid = (NPAD // bn,)
    in_specs = (
        [pl.BlockSpec((bn, 128), lambda i: (i, 0)) for _ in range(nk)]
        + [pl.BlockSpec((bn, 1), lambda i: (i, 0))]
        + [pl.BlockSpec((bn, 128), lambda i: (i, 0)) for _ in range(nj)]
        + [pl.BlockSpec((Wa.shape[0], dout), lambda i: (0, 0)),
           pl.BlockSpec((Wr.shape[0], dout), lambda i: (0, 0)),
           pl.BlockSpec((1, dout), lambda i: (0, 0))]
    )
    out_specs = [pl.BlockSpec((bn, 128), lambda i: (i, 0)) for _ in range(nm)]
    out_shape = [jax.ShapeDtypeStruct((NPAD, 128), jnp.float32)
                 for _ in range(nm)]
    return pl.pallas_call(
        functools.partial(_conv_body, nk, nj, nm),
        grid=grid,
        in_specs=in_specs,
        out_specs=out_specs,
        out_shape=out_shape,
    )(*agg_chunks, cnt, *h_chunks, Wa, Wr, b[None, :])


def _proj_body(nj, nm, *refs):
    hs = refs[0:nj]
    w_ref = refs[nj]
    outs = refs[nj + 1:]
    acc = jnp.dot(hs[0][...], w_ref[pl.ds(0, 128), :],
                  preferred_element_type=jnp.float32)
    for j in range(1, nj):
        acc = acc + jnp.dot(hs[j][...], w_ref[pl.ds(j * 128, 128), :],
                            preferred_element_type=jnp.float32)
    for m, o in enumerate(outs):
        o[...] = acc[:, m * 128:(m + 1) * 128]


def _proj_tc(h_chunks, W):
    nj = len(h_chunks)
    dout = W.shape[1]
    nm = dout // 128
    bn = 1024
    in_specs = ([pl.BlockSpec((bn, 128), lambda i: (i, 0)) for _ in range(nj)]
                + [pl.BlockSpec((W.shape[0], dout), lambda i: (0, 0))])
    return pl.pallas_call(
        functools.partial(_proj_body, nj, nm),
        grid=(NPAD // bn,),
        in_specs=in_specs,
        out_specs=[pl.BlockSpec((bn, 128), lambda i: (i, 0)) for _ in range(nm)],
        out_shape=[jax.ShapeDtypeStruct((NPAD, 128), jnp.float32)
                   for _ in range(nm)],
    )(*h_chunks, W)


def _conv_pre_body(nk, nj, *refs):
    aggs = refs[0:nk]
    cnt_ref = refs[nk]
    hs = refs[nk + 1: nk + 1 + nj]
    wr_ref = refs[nk + 1 + nj]
    b_ref = refs[nk + 2 + nj]
    outs = refs[nk + 3 + nj:]
    inv = 1.0 / jnp.maximum(cnt_ref[...], 1.0)
    acc = jnp.broadcast_to(b_ref[...], (hs[0].shape[0], b_ref.shape[1]))
    for j, h in enumerate(hs):
        acc = acc + jnp.dot(h[...], wr_ref[pl.ds(j * 128, 128), :],
                            preferred_element_type=jnp.float32)
    for m, o in enumerate(outs):
        o[...] = aggs[m][...] * inv + acc[:, m * 128:(m + 1) * 128]


def _conv_pre_tc(agg_chunks, cnt, h_chunks, Wr, b):
    nk, nj = len(agg_chunks), len(h_chunks)
    dout = Wr.shape[1]
    nm = dout // 128
    assert nk == nm
    bn = 1024
    in_specs = (
        [pl.BlockSpec((bn, 128), lambda i: (i, 0)) for _ in range(nk)]
        + [pl.BlockSpec((bn, 1), lambda i: (i, 0))]
        + [pl.BlockSpec((bn, 128), lambda i: (i, 0)) for _ in range(nj)]
        + [pl.BlockSpec((Wr.shape[0], dout), lambda i: (0, 0)),
           pl.BlockSpec((1, dout), lambda i: (0, 0))]
    )
    return pl.pallas_call(
        functools.partial(_conv_pre_body, nk, nj),
        grid=(NPAD // bn,),
        in_specs=in_specs,
        out_specs=[pl.BlockSpec((bn, 128), lambda i: (i, 0)) for _ in range(nm)],
        out_shape=[jax.ShapeDtypeStruct((NPAD, 128), jnp.float32)
                   for _ in range(nm)],
    )(*agg_chunks, cnt, *h_chunks, Wr, b[None, :])


def _linkpred_body(s0, s1, s2, s3, d0, d1, d2, d3, o_ref):
    acc = jnp.sum(s0[...] * d0[...], axis=-1)
    acc += jnp.sum(s1[...] * d1[...], axis=-1)
    acc += jnp.sum(s2[...] * d2[...], axis=-1)
    acc += jnp.sum(s3[...] * d3[...], axis=-1)
    o_ref[...] = jax.nn.sigmoid(acc)


def _linkpred_tc(s_chunks, d_chunks):
    bl = 2048
    spec = pl.BlockSpec((bl, 128), lambda i: (i, 0))
    out = pl.pallas_call(
        _linkpred_body,
        grid=(LPAD // bl,),
        in_specs=[spec] * 8,
        out_specs=pl.BlockSpec((bl,), lambda i: (i,)),
        out_shape=jax.ShapeDtypeStruct((LPAD,), jnp.float32),
    )(*s_chunks, *d_chunks)
    return out[:N_LBL]


def _pad_edges(ei):
    pad = EPAD - E_EDGES
    sidx = jnp.concatenate([ei[0], jnp.arange(pad, dtype=jnp.int32) % N_NODES])
    didx = jnp.concatenate(
        [ei[1], N_NODES + (jnp.arange(pad, dtype=jnp.int32) % 128)])
    return sidx.reshape(EPAD // 128, 128), didx.reshape(EPAD // 128, 128)


def kernel(x_transactions, x_wallets, enc0_Wa, enc0_Wr, enc0_b, enc1_Wa, enc1_Wr, enc1_b,
           dec0_Wa, dec0_Wr, dec0_b, last_Wa, last_Wr, last_b,
           edge_index_tw, edge_index_wt, edge_label_index):
    s_tw, d_tw = _pad_edges(edge_index_tw)
    s_wt, d_wt = _pad_edges(edge_index_wt)
    zeros128 = jnp.zeros((128, 128), jnp.float32)
    ones128 = jnp.ones((128,), jnp.float32)
    zeros640 = jnp.zeros((RPT,), jnp.float32)

    cnt_tw_p, cnt_wt_p = _make_count_kernel()(d_tw, d_wt, ones128, zeros640)
    cnt_tw = cnt_tw_p[:, None]
    cnt_wt = cnt_wt_p[:, None]

    xt_pad = jnp.pad(x_transactions, ((0, NPAD - N_NODES), (0, 0)))
    xw_pad = jnp.pad(x_wallets, ((0, NPAD - N_NODES), (0, 0)))
    ht = [xt_pad[:, i * 128:(i + 1) * 128] for i in range(2)]
    hw = [xw_pad[:, i * 128:(i + 1) * 128] for i in range(2)]

    agg2 = _make_layer_agg(2)
    agg4 = _make_layer_agg(4)

    def layer(ht, hw, Wa, Wr, b):
        aggk = agg2 if len(ht) == 2 else agg4
        res = aggk(s_tw, d_tw, s_wt, d_wt, *ht, *hw, zeros128)
        nch = len(ht)
        aw, at = list(res[:nch]), list(res[nch:])
        new_w = _conv_tc(aw, cnt_tw, hw, Wa[0], Wr[0], b[0])
        new_t = _conv_tc(at, cnt_wt, ht, Wa[1], Wr[1], b[1])
        return list(new_t), list(new_w)

    ht, hw = layer(ht, hw, enc0_Wa, enc0_Wr, enc0_b)
    ht, hw = layer(ht, hw, enc1_Wa, enc1_Wr, enc1_b)
    hid_t_c, hid_w_c = ht, hw
    ft, fw = layer(ht, hw, dec0_Wa, dec0_Wr, dec0_b)
    pt = _proj_tc(ft, last_Wa[0])
    pw = _proj_tc(fw, last_Wa[1])
    res = agg2(s_tw, d_tw, s_wt, d_wt, *pt, *pw, zeros128)
    aw, at = list(res[:2]), list(res[2:])
    new_w = _conv_pre_tc(aw, cnt_tw, fw, last_Wr[0], last_b[0])
    new_t = _conv_pre_tc(at, cnt_wt, ft, last_Wr[1], last_b[1])
    ft, fw = list(new_t), list(new_w)

    eli0 = jnp.concatenate(
        [edge_label_index[0], jnp.zeros((LPAD - N_LBL,), jnp.int32)]
    ).reshape(NUM_TILES, LNB, 128)
    eli1 = jnp.concatenate(
        [edge_label_index[1], jnp.zeros((LPAD - N_LBL,), jnp.int32)]
    ).reshape(NUM_TILES, LNB, 128)
    lp = _make_lp_gather()(eli0, eli1, *hid_t_c, *hid_w_c)
    s_chunks, d_chunks = lp[:4], lp[4:]
    edge_pred = _linkpred_tc(s_chunks, d_chunks)

    hid_t = jnp.concatenate(hid_t_c, axis=1)[:N_NODES]
    hid_w = jnp.concatenate(hid_w_c, axis=1)[:N_NODES]
    f_t = jnp.concatenate(ft, axis=1)[:N_NODES]
    f_w = jnp.concatenate(fw, axis=1)[:N_NODES]
    return (hid_t, hid_w, f_t, f_w, edge_pred)


# speedup vs baseline: 1.1553x; 1.0092x over previous
import functools

import jax
import jax.numpy as jnp
from jax import lax
from jax.experimental import pallas as pl
from jax.experimental.pallas import tpu as pltpu
from jax.experimental.pallas import tpu_sc as plsc

N_NODES = 10000
E_EDGES = 160000
N_LBL = 20000

NUM_TILES = 16
EPAD = 163840
EPT = EPAD // NUM_TILES
NB = EPT // 128
RPAD = 10240
NPAD = RPAD
RPT = RPAD // NUM_TILES
LPAD = 20480
LPT = LPAD // NUM_TILES
LNB = LPT // 128

_MESH = dict(core_axis_name="c", subcore_axis_name="s")


NHB = NB // 2


def _agg_side(sidx2d, didx2d, tables, outs, zeros_hbm, acc, sidx_v, didx_v,
              rows_a, rows_b, g_a, g_b, s_a, s_b, s):
    for tab, out in zip(tables, outs):
        for k in range(RPT // 128):
            pltpu.sync_copy(zeros_hbm, acc.at[pl.ds(s * RPT + k * 128, 128)])
        plsc.subcore_barrier()
        for h in range(2):
            pltpu.sync_copy(sidx2d.at[pl.ds(s * NB + h * NHB, NHB)], sidx_v)
            pltpu.sync_copy(didx2d.at[pl.ds(s * NB + h * NHB, NHB)], didx_v)
            pltpu.async_copy(tab.at[sidx_v.at[0]], rows_a, g_a)

            def grp(g, carry):
                for b in range(2):
                    rows_cur, sem_cur = (rows_a, g_a) if b == 0 else (rows_b, g_b)
                    rows_nxt, sem_nxt = (rows_b, g_b) if b == 0 else (rows_a, g_a)
                    i = g * 2 + b
                    inext = i + 1

                    @pl.when(inext < NHB)
                    def _(inext=inext, rows_nxt=rows_nxt, sem_nxt=sem_nxt):
                        pltpu.async_copy(tab.at[sidx_v.at[inext]], rows_nxt, sem_nxt)

                    pltpu.make_async_copy(tab.at[sidx_v.at[0]], rows_cur, sem_cur).wait()
                    pltpu.sync_copy(rows_cur, acc.at[didx_v.at[i]], add=True)
                return carry

            lax.fori_loop(0, NHB // 2, grp, 0)
        plsc.subcore_barrier()
        for k in range(5):
            r = s * RPT + k * 128
            pltpu.sync_copy(acc.at[pl.ds(r, 128)], rows_a)
            pltpu.sync_copy(rows_a, out.at[pl.ds(r, 128)])
        plsc.subcore_barrier()


def _make_layer_agg(nch):
    mesh = plsc.VectorSubcoreMesh(**_MESH)
    out_type = tuple(jax.ShapeDtypeStruct((NPAD, 128), jnp.float32)
                     for _ in range(2 * nch))
    scratch = [
        pltpu.VMEM_SHARED((RPAD, 128), jnp.float32),
        pltpu.VMEM((NHB, 128), jnp.int32),
        pltpu.VMEM((NHB, 128), jnp.int32),
        pltpu.VMEM((128, 128), jnp.float32),
        pltpu.VMEM((128, 128), jnp.float32),
        pltpu.SemaphoreType.DMA,
        pltpu.SemaphoreType.DMA,
        pltpu.SemaphoreType.DMA,
        pltpu.SemaphoreType.DMA,
    ]

    @functools.partial(pl.kernel, out_type=out_type, mesh=mesh,
                       scratch_types=scratch)
    def k(s_tw, d_tw, s_wt, d_wt, *rest):
        tabs_t = rest[0:nch]
        tabs_w = rest[nch:2 * nch]
        zeros_hbm = rest[2 * nch]
        outs_w = rest[2 * nch + 1: 2 * nch + 1 + nch]
        outs_t = rest[2 * nch + 1 + nch: 2 * nch + 1 + 2 * nch]
        (acc, sidx_v, didx_v, rows_a, rows_b, g_a, g_b, s_a, s_b) = \
            rest[2 * nch + 1 + 2 * nch:]
        c = lax.axis_index("c")
        s = lax.axis_index("s")

        @pl.when(c == 0)
        def _():
            _agg_side(s_tw, d_tw, tabs_t, outs_w, zeros_hbm, acc, sidx_v,
                      didx_v, rows_a, rows_b, g_a, g_b, s_a, s_b, s)

        @pl.when(c == 1)
        def _():
            _agg_side(s_wt, d_wt, tabs_w, outs_t, zeros_hbm, acc, sidx_v,
                      didx_v, rows_a, rows_b, g_a, g_b, s_a, s_b, s)

    return k


def _make_count_kernel():
    mesh = plsc.VectorSubcoreMesh(**_MESH)
    out_type = (jax.ShapeDtypeStruct((RPAD,), jnp.float32),
                jax.ShapeDtypeStruct((RPAD,), jnp.float32))
    scratch = [
        pltpu.VMEM_SHARED((RPAD,), jnp.float32),
        pltpu.VMEM((NB, 128), jnp.int32),
        pltpu.VMEM((128,), jnp.float32),
        pltpu.VMEM((RPT,), jnp.float32),
    ]

    @functools.partial(pl.kernel, out_type=out_type, mesh=mesh,
                       scratch_types=scratch)
    def k(d_tw, d_wt, ones_hbm, zeros_hbm, out_tw, out_wt, acc1, didx_v,
          ones_v, stage_v):
        c = lax.axis_index("c")
        s = lax.axis_index("s")

        def side(didx2d, out):
            pltpu.sync_copy(zeros_hbm, acc1.at[pl.ds(s * RPT, RPT)])
            plsc.subcore_barrier()
            pltpu.sync_copy(didx2d.at[pl.ds(s * NB, NB)], didx_v)
            pltpu.sync_copy(ones_hbm, ones_v)

            def it(i, carry):
                pltpu.sync_copy(ones_v, acc1.at[didx_v.at[i]], add=True)
                return carry

            lax.fori_loop(0, NB, it, 0)
            plsc.subcore_barrier()
            pltpu.sync_copy(acc1.at[pl.ds(s * RPT, RPT)], stage_v)
            pltpu.sync_copy(stage_v, out.at[pl.ds(s * RPT, RPT)])

        @pl.when(c == 0)
        def _():
            side(d_tw, out_tw)

        @pl.when(c == 1)
        def _():
            side(d_wt, out_wt)

    return k


def _make_lp_gather():
    nch = 4
    mesh = plsc.VectorSubcoreMesh(**_MESH)
    out_type = tuple(jax.ShapeDtypeStruct((LPAD, 128), jnp.float32)
                     for _ in range(2 * nch))
    scratch = [
        pltpu.VMEM((LNB, 128), jnp.int32),
        pltpu.VMEM((128, 128), jnp.float32),
        pltpu.VMEM((128, 128), jnp.float32),
        pltpu.SemaphoreType.DMA,
        pltpu.SemaphoreType.DMA,
        pltpu.SemaphoreType.DMA,
        pltpu.SemaphoreType.DMA,
    ]

    @functools.partial(pl.kernel, out_type=out_type, mesh=mesh,
                       scratch_types=scratch)
    def k(eli0, eli1, *rest):
        tabs_t = rest[0:nch]
        tabs_w = rest[nch:2 * nch]
        outs_s = rest[2 * nch: 3 * nch]
        outs_d = rest[3 * nch: 4 * nch]
        idx_v, rows_a, rows_b, g_a, g_b, w_a, w_b = rest[4 * nch:]
        c = lax.axis_index("c")
        s = lax.axis_index("s")

        def side(eli, tabs, outs):
            pltpu.sync_copy(eli.at[s], idx_v)
            for tab, out in zip(tabs, outs):
                pltpu.async_copy(tab.at[idx_v.at[0]], rows_a, g_a)

                def grp(g, carry, tab=tab, out=out):
                    for b in range(2):
                        rows_cur, sem_cur = (rows_a, g_a) if b == 0 else (rows_b, g_b)
                        rows_nxt, sem_nxt = (rows_b, g_b) if b == 0 else (rows_a, g_a)
                        i = g * 2 + b
                        inext = i + 1

                        @pl.when(inext < LNB)
                        def _(inext=inext, rows_nxt=rows_nxt, sem_nxt=sem_nxt):
                            pltpu.async_copy(tab.at[idx_v.at[inext]], rows_nxt, sem_nxt)

                        pltpu.make_async_copy(tab.at[idx_v.at[0]], rows_cur, sem_cur).wait()
                        pltpu.sync_copy(rows_cur, out.at[pl.ds(s * LPT + i * 128, 128)])
                    return carry

                lax.fori_loop(0, LNB // 2, grp, 0)

        @pl.when(c == 0)
        def _():
            side(eli0, tabs_t, outs_s)

        @pl.when(c == 1)
        def _():
            side(eli1, tabs_w, outs_d)

    return k


def _conv_body(nk, nj, nm, *refs):
    aggs = refs[0:nk]
    cnt_ref = refs[nk]
    hs = refs[nk + 1: nk + 1 + nj]
    wa_ref = refs[nk + 1 + nj]
    wr_ref = refs[nk + 2 + nj]
    b_ref = refs[nk + 3 + nj]
    outs = refs[nk + 4 + nj:]
    inv = 1.0 / jnp.maximum(cnt_ref[...], 1.0)
    acc = jnp.broadcast_to(b_ref[...], (aggs[0].shape[0], b_ref.shape[1]))
    for j, a in enumerate(aggs):
        acc = acc + jnp.dot(a[...] * inv, wa_ref[pl.ds(j * 128, 128), :],
                            preferred_element_type=jnp.float32)
    for j, h in enumerate(hs):
        acc = acc + jnp.dot(h[...], wr_ref[pl.ds(j * 128, 128), :],
                            preferred_element_type=jnp.float32)
    for m, o in enumerate(outs):
        o[...] = acc[:, m * 128:(m + 1) * 128]


def _conv_tc(agg_chunks, cnt, h_chunks, Wa, Wr, b):
    nk, nj = len(agg_chunks), len(h_chunks)
    dout = Wa.shape[1]
    nm = dout // 128
    bn = 1024
    grid = (NPAD // bn,)
    in_specs = (
        [pl.BlockSpec((bn, 128), lambda i: (i, 0)) for _ in range(nk)]
        + [pl.BlockSpec((bn, 1), lambda i: (i, 0))]
        + [pl.BlockSpec((bn, 128), lambda i: (i, 0)) for _ in range(nj)]
        + [pl.BlockSpec((Wa.shape[0], dout), lambda i: (0, 0)),
           pl.BlockSpec((Wr.shape[0], dout), lambda i: (0, 0)),
           pl.BlockSpec((1, dout), lambda i: (0, 0))]
    )
    out_specs = [pl.BlockSpec((bn, 128), lambda i: (i, 0)) for _ in range(nm)]
    out_shape = [jax.ShapeDtypeStruct((NPAD, 128), jnp.float32)
                 for _ in range(nm)]
    return pl.pallas_call(
        functools.partial(_conv_body, nk, nj, nm),
        grid=grid,
        in_specs=in_specs,
        out_specs=out_specs,
        out_shape=out_shape,
    )(*agg_chunks, cnt, *h_chunks, Wa, Wr, b[None, :])


def _proj_body(nj, has_b, *refs):
    hs = refs[0:nj]
    w_ref = refs[nj]
    b_ref = refs[nj + 1] if has_b else None
    outs = refs[nj + 1 + int(has_b):]
    acc = jnp.dot(hs[0][...], w_ref[pl.ds(0, 128), :],
                  preferred_element_type=jnp.float32)
    for j in range(1, nj):
        acc = acc + jnp.dot(hs[j][...], w_ref[pl.ds(j * 128, 128), :],
                            preferred_element_type=jnp.float32)
    if has_b:
        acc = acc + b_ref[...]
    for m, o in enumerate(outs):
        o[...] = acc[:, m * 128:(m + 1) * 128]


def _proj_tc(h_chunks, W, b=None):
    nj = len(h_chunks)
    dout = W.shape[1]
    nm = dout // 128
    bn = 1024
    in_specs = ([pl.BlockSpec((bn, 128), lambda i: (i, 0)) for _ in range(nj)]
                + [pl.BlockSpec((W.shape[0], dout), lambda i: (0, 0))])
    args = list(h_chunks) + [W]
    if b is not None:
        in_specs = in_specs + [pl.BlockSpec((1, dout), lambda i: (0, 0))]
        args = args + [b[None, :]]
    return pl.pallas_call(
        functools.partial(_proj_body, nj, b is not None),
        grid=(NPAD // bn,),
        in_specs=in_specs,
        out_specs=[pl.BlockSpec((bn, 128), lambda i: (i, 0)) for _ in range(nm)],
        out_shape=[jax.ShapeDtypeStruct((NPAD, 128), jnp.float32)
                   for _ in range(nm)],
    )(*args)


def _combine_body(nk, *refs):
    aggs = refs[0:nk]
    cnt_ref = refs[nk]
    wa_ref = refs[nk + 1]
    nm = (len(refs) - nk - 2) // 2
    selfs = refs[nk + 2: nk + 2 + nm]
    outs = refs[nk + 2 + nm:]
    inv = 1.0 / jnp.maximum(cnt_ref[...], 1.0)
    acc = jnp.dot(aggs[0][...] * inv, wa_ref[pl.ds(0, 128), :],
                  preferred_element_type=jnp.float32)
    for j in range(1, nk):
        acc = acc + jnp.dot(aggs[j][...] * inv, wa_ref[pl.ds(j * 128, 128), :],
                            preferred_element_type=jnp.float32)
    for m, o in enumerate(outs):
        o[...] = acc[:, m * 128:(m + 1) * 128] + selfs[m][...]


def _combine_tc(agg_chunks, cnt, self_chunks, Wa):
    nk, nm = len(agg_chunks), len(self_chunks)
    dout = Wa.shape[1]
    assert nm == dout // 128
    bn = 1024
    in_specs = (
        [pl.BlockSpec((bn, 128), lambda i: (i, 0)) for _ in range(nk)]
        + [pl.BlockSpec((bn, 1), lambda i: (i, 0))]
        + [pl.BlockSpec((Wa.shape[0], dout), lambda i: (0, 0))]
        + [pl.BlockSpec((bn, 128), lambda i: (i, 0)) for _ in range(nm)]
    )
    return pl.pallas_call(
        functools.partial(_combine_body, nk),
        grid=(NPAD // bn,),
        in_specs=in_specs,
        out_specs=[pl.BlockSpec((bn, 128), lambda i: (i, 0)) for _ in range(nm)],
        out_shape=[jax.ShapeDtypeStruct((NPAD, 128), jnp.float32)
                   for _ in range(nm)],
    )(*agg_chunks, cnt, Wa, *self_chunks)


def _combine_pre_body(nm, *refs):
    aggs = refs[0:nm]
    cnt_ref = refs[nm]
    selfs = refs[nm + 1: 2 * nm + 1]
    outs = refs[2 * nm + 1:]
    inv = 1.0 / jnp.maximum(cnt_ref[...], 1.0)
    for m, o in enumerate(outs):
        o[...] = aggs[m][...] * inv + selfs[m][...]


def _combine_pre_tc(agg_chunks, cnt, self_chunks):
    nm = len(agg_chunks)
    bn = 1024
    in_specs = (
        [pl.BlockSpec((bn, 128), lambda i: (i, 0)) for _ in range(nm)]
        + [pl.BlockSpec((bn, 1), lambda i: (i, 0))]
        + [pl.BlockSpec((bn, 128), lambda i: (i, 0)) for _ in range(nm)]
    )
    return pl.pallas_call(
        functools.partial(_combine_pre_body, nm),
        grid=(NPAD // bn,),
        in_specs=in_specs,
        out_specs=[pl.BlockSpec((bn, 128), lambda i: (i, 0)) for _ in range(nm)],
        out_shape=[jax.ShapeDtypeStruct((NPAD, 128), jnp.float32)
                   for _ in range(nm)],
    )(*agg_chunks, cnt, *self_chunks)


def _linkpred_body(s0, s1, s2, s3, d0, d1, d2, d3, o_ref):
    acc = jnp.sum(s0[...] * d0[...], axis=-1)
    acc += jnp.sum(s1[...] * d1[...], axis=-1)
    acc += jnp.sum(s2[...] * d2[...], axis=-1)
    acc += jnp.sum(s3[...] * d3[...], axis=-1)
    o_ref[...] = jax.nn.sigmoid(acc)


def _linkpred_tc(s_chunks, d_chunks):
    bl = 2048
    spec = pl.BlockSpec((bl, 128), lambda i: (i, 0))
    out = pl.pallas_call(
        _linkpred_body,
        grid=(LPAD // bl,),
        in_specs=[spec] * 8,
        out_specs=pl.BlockSpec((bl,), lambda i: (i,)),
        out_shape=jax.ShapeDtypeStruct((LPAD,), jnp.float32),
    )(*s_chunks, *d_chunks)
    return out[:N_LBL]


def _pad_edges(ei):
    pad = EPAD - E_EDGES
    sidx = jnp.concatenate([ei[0], jnp.arange(pad, dtype=jnp.int32) % N_NODES])
    didx = jnp.concatenate(
        [ei[1], N_NODES + (jnp.arange(pad, dtype=jnp.int32) % 128)])
    return sidx.reshape(EPAD // 128, 128), didx.reshape(EPAD // 128, 128)


def kernel(x_transactions, x_wallets, enc0_Wa, enc0_Wr, enc0_b, enc1_Wa, enc1_Wr, enc1_b,
           dec0_Wa, dec0_Wr, dec0_b, last_Wa, last_Wr, last_b,
           edge_index_tw, edge_index_wt, edge_label_index):
    s_tw, d_tw = _pad_edges(edge_index_tw)
    s_wt, d_wt = _pad_edges(edge_index_wt)
    zeros128 = jnp.zeros((128, 128), jnp.float32)
    ones128 = jnp.ones((128,), jnp.float32)
    zeros640 = jnp.zeros((RPT,), jnp.float32)

    cnt_tw_p, cnt_wt_p = _make_count_kernel()(d_tw, d_wt, ones128, zeros640)
    cnt_tw = cnt_tw_p[:, None]
    cnt_wt = cnt_wt_p[:, None]

    xt_pad = jnp.pad(x_transactions, ((0, NPAD - N_NODES), (0, 0)))
    xw_pad = jnp.pad(x_wallets, ((0, NPAD - N_NODES), (0, 0)))
    ht = [xt_pad[:, i * 128:(i + 1) * 128] for i in range(2)]
    hw = [xw_pad[:, i * 128:(i + 1) * 128] for i in range(2)]

    agg2 = _make_layer_agg(2)
    agg4 = _make_layer_agg(4)

    def layer(ht, hw, Wa, Wr, b):
        aggk = agg2 if len(ht) == 2 else agg4
        res = aggk(s_tw, d_tw, s_wt, d_wt, *ht, *hw, zeros128)
        self_w = _proj_tc(hw, Wr[0], b[0])
        self_t = _proj_tc(ht, Wr[1], b[1])
        nch = len(ht)
        aw, at = list(res[:nch]), list(res[nch:])
        new_w = _combine_tc(aw, cnt_tw, self_w, Wa[0])
        new_t = _combine_tc(at, cnt_wt, self_t, Wa[1])
        return list(new_t), list(new_w)

    eli0 = jnp.concatenate(
        [edge_label_index[0], jnp.zeros((LPAD - N_LBL,), jnp.int32)]
    ).reshape(NUM_TILES, LNB, 128)
    eli1 = jnp.concatenate(
        [edge_label_index[1], jnp.zeros((LPAD - N_LBL,), jnp.int32)]
    ).reshape(NUM_TILES, LNB, 128)

    ht, hw = layer(ht, hw, enc0_Wa, enc0_Wr, enc0_b)
    ht, hw = layer(ht, hw, enc1_Wa, enc1_Wr, enc1_b)
    hid_t_c, hid_w_c = ht, hw

    lp = _make_lp_gather()(eli0, eli1, *hid_t_c, *hid_w_c)
    s_chunks, d_chunks = lp[:4], lp[4:]
    edge_pred = _linkpred_tc(s_chunks, d_chunks)

    ft, fw = layer(ht, hw, dec0_Wa, dec0_Wr, dec0_b)
    pt = _proj_tc(ft, last_Wa[0])
    pw = _proj_tc(fw, last_Wa[1])
    res = agg2(s_tw, d_tw, s_wt, d_wt, *pt, *pw, zeros128)
    self_w = _proj_tc(fw, last_Wr[0], last_b[0])
    self_t = _proj_tc(ft, last_Wr[1], last_b[1])
    aw, at = list(res[:2]), list(res[2:])
    fw = list(_combine_pre_tc(aw, cnt_tw, self_w))
    ft = list(_combine_pre_tc(at, cnt_wt, self_t))

    hid_t = jnp.concatenate(hid_t_c, axis=1)[:N_NODES]
    hid_w = jnp.concatenate(hid_w_c, axis=1)[:N_NODES]
    f_t = jnp.concatenate(ft, axis=1)[:N_NODES]
    f_w = jnp.concatenate(fw, axis=1)[:N_NODES]
    return (hid_t, hid_w, f_t, f_w, edge_pred)


# speedup vs baseline: 1.2787x; 1.1069x over previous
import functools

import jax
import jax.numpy as jnp
from jax import lax
from jax.experimental import pallas as pl
from jax.experimental.pallas import tpu as pltpu
from jax.experimental.pallas import tpu_sc as plsc

N_NODES = 10000
E_EDGES = 160000
N_LBL = 20000

NUM_TILES = 16
EPAD = 163840
EPT = EPAD // NUM_TILES
NB = EPT // 128
RPAD = 10240
NPAD = RPAD
RPT = RPAD // NUM_TILES
LPAD = 20480
LPT = LPAD // NUM_TILES
LNB = LPT // 128

_MESH = dict(core_axis_name="c", subcore_axis_name="s")


NHB = NB // 2


def _agg_side(sidx2d, didx2d, tables, outs, zeros_hbm, acc, sidx_v, didx_v,
              rows_a, rows_b, g_a, g_b, s_a, s_b, s):
    for tab, out in zip(tables, outs):
        pltpu.sync_copy(zeros_hbm, rows_a)
        for k in range(RPT // 128):
            pltpu.sync_copy(rows_a, acc.at[pl.ds(s * RPT + k * 128, 128)])
        plsc.subcore_barrier()
        for h in range(2):
            pltpu.sync_copy(sidx2d.at[pl.ds(s * NB + h * NHB, NHB)], sidx_v)
            pltpu.sync_copy(didx2d.at[pl.ds(s * NB + h * NHB, NHB)], didx_v)
            pltpu.async_copy(tab.at[sidx_v.at[0]], rows_a, g_a)

            def grp(g, carry):
                for b in range(2):
                    rows_cur, sem_cur = (rows_a, g_a) if b == 0 else (rows_b, g_b)
                    rows_nxt, sem_nxt = (rows_b, g_b) if b == 0 else (rows_a, g_a)
                    i = g * 2 + b
                    inext = i + 1

                    @pl.when(inext < NHB)
                    def _(inext=inext, rows_nxt=rows_nxt, sem_nxt=sem_nxt):
                        pltpu.async_copy(tab.at[sidx_v.at[inext]], rows_nxt, sem_nxt)

                    pltpu.make_async_copy(tab.at[sidx_v.at[0]], rows_cur, sem_cur).wait()
                    pltpu.sync_copy(rows_cur, acc.at[didx_v.at[i]], add=True)
                return carry

            lax.fori_loop(0, NHB // 2, grp, 0)
        plsc.subcore_barrier()
        for k in range(5):
            r = s * RPT + k * 128
            pltpu.sync_copy(acc.at[pl.ds(r, 128)], rows_a)
            pltpu.sync_copy(rows_a, out.at[pl.ds(r, 128)])
        plsc.subcore_barrier()


def _make_layer_agg(nch):
    mesh = plsc.VectorSubcoreMesh(**_MESH)
    out_type = tuple(jax.ShapeDtypeStruct((NPAD, 128), jnp.float32)
                     for _ in range(2 * nch))
    scratch = [
        pltpu.VMEM_SHARED((RPAD, 128), jnp.float32),
        pltpu.VMEM((NHB, 128), jnp.int32),
        pltpu.VMEM((NHB, 128), jnp.int32),
        pltpu.VMEM((128, 128), jnp.float32),
        pltpu.VMEM((128, 128), jnp.float32),
        pltpu.SemaphoreType.DMA,
        pltpu.SemaphoreType.DMA,
        pltpu.SemaphoreType.DMA,
        pltpu.SemaphoreType.DMA,
    ]

    @functools.partial(pl.kernel, out_type=out_type, mesh=mesh,
                       scratch_types=scratch)
    def k(s_tw, d_tw, s_wt, d_wt, *rest):
        tabs_t = rest[0:nch]
        tabs_w = rest[nch:2 * nch]
        zeros_hbm = rest[2 * nch]
        outs_w = rest[2 * nch + 1: 2 * nch + 1 + nch]
        outs_t = rest[2 * nch + 1 + nch: 2 * nch + 1 + 2 * nch]
        (acc, sidx_v, didx_v, rows_a, rows_b, g_a, g_b, s_a, s_b) = \
            rest[2 * nch + 1 + 2 * nch:]
        c = lax.axis_index("c")
        s = lax.axis_index("s")

        @pl.when(c == 0)
        def _():
            _agg_side(s_tw, d_tw, tabs_t, outs_w, zeros_hbm, acc, sidx_v,
                      didx_v, rows_a, rows_b, g_a, g_b, s_a, s_b, s)

        @pl.when(c == 1)
        def _():
            _agg_side(s_wt, d_wt, tabs_w, outs_t, zeros_hbm, acc, sidx_v,
                      didx_v, rows_a, rows_b, g_a, g_b, s_a, s_b, s)

    return k


def _make_count_kernel():
    mesh = plsc.VectorSubcoreMesh(**_MESH)
    out_type = (jax.ShapeDtypeStruct((RPAD,), jnp.float32),
                jax.ShapeDtypeStruct((RPAD,), jnp.float32))
    scratch = [
        pltpu.VMEM_SHARED((RPAD,), jnp.float32),
        pltpu.VMEM((NB, 128), jnp.int32),
        pltpu.VMEM((128,), jnp.float32),
        pltpu.VMEM((RPT,), jnp.float32),
    ]

    @functools.partial(pl.kernel, out_type=out_type, mesh=mesh,
                       scratch_types=scratch)
    def k(d_tw, d_wt, ones_hbm, zeros_hbm, out_tw, out_wt, acc1, didx_v,
          ones_v, stage_v):
        c = lax.axis_index("c")
        s = lax.axis_index("s")

        def side(didx2d, out):
            pltpu.sync_copy(zeros_hbm, acc1.at[pl.ds(s * RPT, RPT)])
            plsc.subcore_barrier()
            pltpu.sync_copy(didx2d.at[pl.ds(s * NB, NB)], didx_v)
            pltpu.sync_copy(ones_hbm, ones_v)

            def it(i, carry):
                pltpu.sync_copy(ones_v, acc1.at[didx_v.at[i]], add=True)
                return carry

            lax.fori_loop(0, NB, it, 0)
            plsc.subcore_barrier()
            pltpu.sync_copy(acc1.at[pl.ds(s * RPT, RPT)], stage_v)
            pltpu.sync_copy(stage_v, out.at[pl.ds(s * RPT, RPT)])

        @pl.when(c == 0)
        def _():
            side(d_tw, out_tw)

        @pl.when(c == 1)
        def _():
            side(d_wt, out_wt)

    return k


def _make_lp_gather():
    nch = 4
    mesh = plsc.VectorSubcoreMesh(**_MESH)
    out_type = tuple(jax.ShapeDtypeStruct((LPAD, 128), jnp.float32)
                     for _ in range(2 * nch))
    scratch = [
        pltpu.VMEM((LNB, 128), jnp.int32),
        pltpu.VMEM((128, 128), jnp.float32),
        pltpu.VMEM((128, 128), jnp.float32),
        pltpu.VMEM((128, 128), jnp.float32),
        pltpu.VMEM((128, 128), jnp.float32),
        pltpu.SemaphoreType.DMA,
        pltpu.SemaphoreType.DMA,
        pltpu.SemaphoreType.DMA,
        pltpu.SemaphoreType.DMA,
    ]

    @functools.partial(pl.kernel, out_type=out_type, mesh=mesh,
                       scratch_types=scratch)
    def k(eli0, eli1, *rest):
        tabs_t = rest[0:nch]
        tabs_w = rest[nch:2 * nch]
        outs_s = rest[2 * nch: 3 * nch]
        outs_d = rest[3 * nch: 4 * nch]
        idx_v = rest[4 * nch]
        bufs = rest[4 * nch + 1: 4 * nch + 5]
        sems = rest[4 * nch + 5: 4 * nch + 9]
        c = lax.axis_index("c")
        s = lax.axis_index("s")

        def side(eli, tabs, outs):
            pltpu.sync_copy(eli.at[s], idx_v)
            for tab, out in zip(tabs, outs):
                for p in range(3):
                    pltpu.async_copy(tab.at[idx_v.at[p]], bufs[p], sems[p])
                for i in range(LNB):
                    b = i % 4
                    pltpu.make_async_copy(
                        tab.at[idx_v.at[0]], bufs[b], sems[b]).wait()
                    nxt = i + 3
                    if nxt < LNB:
                        pltpu.async_copy(
                            tab.at[idx_v.at[nxt]], bufs[nxt % 4], sems[nxt % 4])
                    pltpu.sync_copy(
                        bufs[b], out.at[pl.ds(s * LPT + i * 128, 128)])

        @pl.when(c == 0)
        def _():
            side(eli0, tabs_t, outs_s)

        @pl.when(c == 1)
        def _():
            side(eli1, tabs_w, outs_d)

    return k


def _conv_body(nk, nj, nm, *refs):
    aggs = refs[0:nk]
    cnt_ref = refs[nk]
    hs = refs[nk + 1: nk + 1 + nj]
    wa_ref = refs[nk + 1 + nj]
    wr_ref = refs[nk + 2 + nj]
    b_ref = refs[nk + 3 + nj]
    outs = refs[nk + 4 + nj:]
    inv = 1.0 / jnp.maximum(cnt_ref[...], 1.0)
    acc = jnp.broadcast_to(b_ref[...], (aggs[0].shape[0], b_ref.shape[1]))
    for j, a in enumerate(aggs):
        acc = acc + jnp.dot(a[...] * inv, wa_ref[pl.ds(j * 128, 128), :],
                            preferred_element_type=jnp.float32)
    for j, h in enumerate(hs):
        acc = acc + jnp.dot(h[...], wr_ref[pl.ds(j * 128, 128), :],
                            preferred_element_type=jnp.float32)
    for m, o in enumerate(outs):
        o[...] = acc[:, m * 128:(m + 1) * 128]


def _conv_tc(agg_chunks, cnt, h_chunks, Wa, Wr, b):
    nk, nj = len(agg_chunks), len(h_chunks)
    dout = Wa.shape[1]
    nm = dout // 128
    bn = 1024
    grid = (NPAD // bn,)
    in_specs = (
        [pl.BlockSpec((bn, 128), lambda i: (i, 0)) for _ in range(nk)]
        + [pl.BlockSpec((bn, 1), lambda i: (i, 0))]
        + [pl.BlockSpec((bn, 128), lambda i: (i, 0)) for _ in range(nj)]
        + [pl.BlockSpec((Wa.shape[0], dout), lambda i: (0, 0)),
           pl.BlockSpec((Wr.shape[0], dout), lambda i: (0, 0)),
           pl.BlockSpec((1, dout), lambda i: (0, 0))]
    )
    out_specs = [pl.BlockSpec((bn, 128), lambda i: (i, 0)) for _ in range(nm)]
    out_shape = [jax.ShapeDtypeStruct((NPAD, 128), jnp.float32)
                 for _ in range(nm)]
    return pl.pallas_call(
        functools.partial(_conv_body, nk, nj, nm),
        grid=grid,
        in_specs=in_specs,
        out_specs=out_specs,
        out_shape=out_shape,
    )(*agg_chunks, cnt, *h_chunks, Wa, Wr, b[None, :])


def _proj_body(nj, has_b, *refs):
    hs = refs[0:nj]
    w_ref = refs[nj]
    b_ref = refs[nj + 1] if has_b else None
    outs = refs[nj + 1 + int(has_b):]
    acc = jnp.dot(hs[0][...], w_ref[pl.ds(0, 128), :],
                  preferred_element_type=jnp.float32)
    for j in range(1, nj):
        acc = acc + jnp.dot(hs[j][...], w_ref[pl.ds(j * 128, 128), :],
                            preferred_element_type=jnp.float32)
    if has_b:
        acc = acc + b_ref[...]
    for m, o in enumerate(outs):
        o[...] = acc[:, m * 128:(m + 1) * 128]


def _proj_tc(h_chunks, W, b=None):
    nj = len(h_chunks)
    dout = W.shape[1]
    nm = dout // 128
    bn = 1024
    in_specs = ([pl.BlockSpec((bn, 128), lambda i: (i, 0)) for _ in range(nj)]
                + [pl.BlockSpec((W.shape[0], dout), lambda i: (0, 0))])
    args = list(h_chunks) + [W]
    if b is not None:
        in_specs = in_specs + [pl.BlockSpec((1, dout), lambda i: (0, 0))]
        args = args + [b[None, :]]
    return pl.pallas_call(
        functools.partial(_proj_body, nj, b is not None),
        grid=(NPAD // bn,),
        in_specs=in_specs,
        out_specs=[pl.BlockSpec((bn, 128), lambda i: (i, 0)) for _ in range(nm)],
        out_shape=[jax.ShapeDtypeStruct((NPAD, 128), jnp.float32)
                   for _ in range(nm)],
    )(*args)


def _combine_body(nk, *refs):
    aggs = refs[0:nk]
    cnt_ref = refs[nk]
    wa_ref = refs[nk + 1]
    nm = (len(refs) - nk - 2) // 2
    selfs = refs[nk + 2: nk + 2 + nm]
    outs = refs[nk + 2 + nm:]
    inv = 1.0 / jnp.maximum(cnt_ref[...], 1.0)
    acc = jnp.dot(aggs[0][...] * inv, wa_ref[pl.ds(0, 128), :],
                  preferred_element_type=jnp.float32)
    for j in range(1, nk):
        acc = acc + jnp.dot(aggs[j][...] * inv, wa_ref[pl.ds(j * 128, 128), :],
                            preferred_element_type=jnp.float32)
    for m, o in enumerate(outs):
        o[...] = acc[:, m * 128:(m + 1) * 128] + selfs[m][...]


def _combine_tc(agg_chunks, cnt, self_chunks, Wa):
    nk, nm = len(agg_chunks), len(self_chunks)
    dout = Wa.shape[1]
    assert nm == dout // 128
    bn = 1024
    in_specs = (
        [pl.BlockSpec((bn, 128), lambda i: (i, 0)) for _ in range(nk)]
        + [pl.BlockSpec((bn, 1), lambda i: (i, 0))]
        + [pl.BlockSpec((Wa.shape[0], dout), lambda i: (0, 0))]
        + [pl.BlockSpec((bn, 128), lambda i: (i, 0)) for _ in range(nm)]
    )
    return pl.pallas_call(
        functools.partial(_combine_body, nk),
        grid=(NPAD // bn,),
        in_specs=in_specs,
        out_specs=[pl.BlockSpec((bn, 128), lambda i: (i, 0)) for _ in range(nm)],
        out_shape=[jax.ShapeDtypeStruct((NPAD, 128), jnp.float32)
                   for _ in range(nm)],
    )(*agg_chunks, cnt, Wa, *self_chunks)


def _combine_pre_body(nm, *refs):
    aggs = refs[0:nm]
    cnt_ref = refs[nm]
    selfs = refs[nm + 1: 2 * nm + 1]
    outs = refs[2 * nm + 1:]
    inv = 1.0 / jnp.maximum(cnt_ref[...], 1.0)
    for m, o in enumerate(outs):
        o[...] = aggs[m][...] * inv + selfs[m][...]


def _combine_pre_tc(agg_chunks, cnt, self_chunks):
    nm = len(agg_chunks)
    bn = 1024
    in_specs = (
        [pl.BlockSpec((bn, 128), lambda i: (i, 0)) for _ in range(nm)]
        + [pl.BlockSpec((bn, 1), lambda i: (i, 0))]
        + [pl.BlockSpec((bn, 128), lambda i: (i, 0)) for _ in range(nm)]
    )
    return pl.pallas_call(
        functools.partial(_combine_pre_body, nm),
        grid=(NPAD // bn,),
        in_specs=in_specs,
        out_specs=[pl.BlockSpec((bn, 128), lambda i: (i, 0)) for _ in range(nm)],
        out_shape=[jax.ShapeDtypeStruct((NPAD, 128), jnp.float32)
                   for _ in range(nm)],
    )(*agg_chunks, cnt, *self_chunks)


def _linkpred_body(s0, s1, s2, s3, d0, d1, d2, d3, o_ref):
    acc = jnp.sum(s0[...] * d0[...], axis=-1)
    acc += jnp.sum(s1[...] * d1[...], axis=-1)
    acc += jnp.sum(s2[...] * d2[...], axis=-1)
    acc += jnp.sum(s3[...] * d3[...], axis=-1)
    o_ref[...] = jax.nn.sigmoid(acc)


def _linkpred_tc(s_chunks, d_chunks):
    bl = 2048
    spec = pl.BlockSpec((bl, 128), lambda i: (i, 0))
    out = pl.pallas_call(
        _linkpred_body,
        grid=(LPAD // bl,),
        in_specs=[spec] * 8,
        out_specs=pl.BlockSpec((bl,), lambda i: (i,)),
        out_shape=jax.ShapeDtypeStruct((LPAD,), jnp.float32),
    )(*s_chunks, *d_chunks)
    return out[:N_LBL]


def _pad_edges(ei):
    pad = EPAD - E_EDGES
    sidx = jnp.concatenate([ei[0], jnp.arange(pad, dtype=jnp.int32) % N_NODES])
    didx = jnp.concatenate(
        [ei[1], N_NODES + (jnp.arange(pad, dtype=jnp.int32) % (RPAD - N_NODES))])
    return sidx.reshape(EPAD // 128, 128), didx.reshape(EPAD // 128, 128)


def kernel(x_transactions, x_wallets, enc0_Wa, enc0_Wr, enc0_b, enc1_Wa, enc1_Wr, enc1_b,
           dec0_Wa, dec0_Wr, dec0_b, last_Wa, last_Wr, last_b,
           edge_index_tw, edge_index_wt, edge_label_index):
    s_tw, d_tw = _pad_edges(edge_index_tw)
    s_wt, d_wt = _pad_edges(edge_index_wt)
    zeros128 = jnp.zeros((128, 128), jnp.float32)
    ones128 = jnp.ones((128,), jnp.float32)
    zeros640 = jnp.zeros((RPT,), jnp.float32)

    cnt_tw_p, cnt_wt_p = _make_count_kernel()(d_tw, d_wt, ones128, zeros640)
    cnt_tw = cnt_tw_p[:, None]
    cnt_wt = cnt_wt_p[:, None]

    xt_pad = jnp.pad(x_transactions, ((0, NPAD - N_NODES), (0, 0)))
    xw_pad = jnp.pad(x_wallets, ((0, NPAD - N_NODES), (0, 0)))
    ht = [xt_pad[:, i * 128:(i + 1) * 128] for i in range(2)]
    hw = [xw_pad[:, i * 128:(i + 1) * 128] for i in range(2)]

    agg2 = _make_layer_agg(2)
    agg4 = _make_layer_agg(4)

    def layer(ht, hw, Wa, Wr, b):
        aggk = agg2 if len(ht) == 2 else agg4
        res = aggk(s_tw, d_tw, s_wt, d_wt, *ht, *hw, zeros128)
        self_w = _proj_tc(hw, Wr[0], b[0])
        self_t = _proj_tc(ht, Wr[1], b[1])
        nch = len(ht)
        aw, at = list(res[:nch]), list(res[nch:])
        new_w = _combine_tc(aw, cnt_tw, self_w, Wa[0])
        new_t = _combine_tc(at, cnt_wt, self_t, Wa[1])
        return list(new_t), list(new_w)

    lpad_fill = jnp.arange(LPAD - N_LBL, dtype=jnp.int32) % N_NODES
    eli0 = jnp.concatenate(
        [edge_label_index[0], lpad_fill]).reshape(NUM_TILES, LNB, 128)
    eli1 = jnp.concatenate(
        [edge_label_index[1], lpad_fill]).reshape(NUM_TILES, LNB, 128)

    ht, hw = layer(ht, hw, enc0_Wa, enc0_Wr, enc0_b)
    ht, hw = layer(ht, hw, enc1_Wa, enc1_Wr, enc1_b)
    hid_t_c, hid_w_c = ht, hw

    lp = _make_lp_gather()(eli0, eli1, *hid_t_c, *hid_w_c)
    s_chunks, d_chunks = lp[:4], lp[4:]
    edge_pred = _linkpred_tc(s_chunks, d_chunks)

    ft, fw = layer(ht, hw, dec0_Wa, dec0_Wr, dec0_b)
    pt = _proj_tc(ft, last_Wa[0])
    pw = _proj_tc(fw, last_Wa[1])
    res = agg2(s_tw, d_tw, s_wt, d_wt, *pt, *pw, zeros128)
    self_w = _proj_tc(fw, last_Wr[0], last_b[0])
    self_t = _proj_tc(ft, last_Wr[1], last_b[1])
    aw, at = list(res[:2]), list(res[2:])
    fw = list(_combine_pre_tc(aw, cnt_tw, self_w))
    ft = list(_combine_pre_tc(at, cnt_wt, self_t))

    hid_t = jnp.concatenate(hid_t_c, axis=1)[:N_NODES]
    hid_w = jnp.concatenate(hid_w_c, axis=1)[:N_NODES]
    f_t = jnp.concatenate(ft, axis=1)[:N_NODES]
    f_w = jnp.concatenate(fw, axis=1)[:N_NODES]
    return (hid_t, hid_w, f_t, f_w, edge_pred)
